# baseline (speedup 1.0000x reference)
import functools

import jax
import jax.numpy as jnp
import numpy as np
from jax import lax
from jax.experimental import pallas as pl
from jax.experimental.pallas import tpu as pltpu
from jax.experimental.pallas import tpu_sc as plsc

VOCAB = 1000000
D = 64
HID = 128
NCLS = 2
B = 4096
L = 200

NW = 32
SPW = B // NW
IPW = SPW * L
CHUNK = 128
NCHUNK = IPW // CHUNK

_mesh = plsc.VectorSubcoreMesh(core_axis_name="c", subcore_axis_name="s")


@functools.partial(
    pl.kernel,
    out_type=jax.ShapeDtypeStruct((B, D), jnp.float32),
    mesh=_mesh,
    scratch_types=[
        pltpu.VMEM((IPW,), jnp.int32),
        pltpu.VMEM((NCHUNK, CHUNK), jnp.int32),
        pltpu.VMEM((CHUNK, D), jnp.float32),
        pltpu.VMEM_SHARED((16 * SPW, D), jnp.float32),
        pltpu.SemaphoreType.DMA,
    ],
    compiler_params=pltpu.CompilerParams(use_tc_tiling_on_sc=False),
)
def _sc_pool(x_hbm, seg_hbm, zero_hbm, emb_hbm, out_hbm,
             idx_v, seg_v, rows_v, acc_sh, sem):
    sid = lax.axis_index("s")
    wid = sid * 2 + lax.axis_index("c")
    base = wid * IPW
    pltpu.sync_copy(x_hbm.at[pl.ds(base, IPW)], idx_v)
    pltpu.sync_copy(seg_hbm.at[sid], seg_v)
    pltpu.sync_copy(zero_hbm, acc_sh.at[pl.ds(sid * SPW, SPW)])

    @pl.loop(0, NCHUNK)
    def _chunk(c):
        off = pl.multiple_of(c * CHUNK, CHUNK)
        pltpu.async_copy(emb_hbm.at[idx_v.at[pl.ds(off, CHUNK)]],
                         rows_v, sem).wait()
        pltpu.sync_copy(rows_v, acc_sh.at[seg_v.at[c]], add=True)

    pltpu.sync_copy(acc_sh.at[pl.ds(sid * SPW, SPW)],
                    out_hbm.at[pl.ds(wid * SPW, SPW)])


def _mlp_body(s_ref, w1_ref, b1_ref, w2_ref, b2_ref, o_ref):
    x = s_ref[...] * np.float32(1.0 / L)
    h = lax.dot_general(x, w1_ref[...], (((1,), (1,)), ((), ())),
                        preferred_element_type=jnp.float32)
    h = jnp.maximum(h + b1_ref[...], 0.0)
    o_ref[...] = lax.dot_general(h, w2_ref[...], (((1,), (1,)), ((), ())),
                                 preferred_element_type=jnp.float32) + b2_ref[...]


def _mlp(sums, W1, b1, W2, b2):
    return pl.pallas_call(
        _mlp_body,
        out_shape=jax.ShapeDtypeStruct((B, NCLS), jnp.float32),
    )(sums, W1, b1.reshape(1, HID), W2, b2.reshape(1, NCLS))


_SEG = (np.arange(16, dtype=np.int32)[:, None] * SPW
        + np.arange(IPW, dtype=np.int32)[None, :] // L
        ).reshape(16, NCHUNK, CHUNK)


def kernel(x_in, emb, W1, b1, W2, b2):
    x_flat = x_in.reshape(-1)
    seg = jnp.asarray(_SEG)
    zero = jnp.zeros((SPW, D), jnp.float32)
    sums = _sc_pool(x_flat, seg, zero, emb)
    return _mlp(sums, W1, b1, W2, b2)


# speedup vs baseline: 1.2879x; 1.2879x over previous
import functools

import jax
import jax.numpy as jnp
import numpy as np
from jax import lax
from jax.experimental import pallas as pl
from jax.experimental.pallas import tpu as pltpu
from jax.experimental.pallas import tpu_sc as plsc

VOCAB = 1000000
D = 64
HID = 128
NCLS = 2
B = 4096
L = 200

NW = 32
SPW = B // NW
WINDOW = 16

_mesh = plsc.VectorSubcoreMesh(core_axis_name="c", subcore_axis_name="s")


@functools.partial(
    pl.kernel,
    out_type=jax.ShapeDtypeStruct((B, D), jnp.float32),
    mesh=_mesh,
    scratch_types=[
        pltpu.VMEM((L, SPW), jnp.int32),
        pltpu.VMEM((SPW, D), jnp.float32),
        pltpu.SemaphoreType.DMA,
    ],
    compiler_params=pltpu.CompilerParams(use_tc_tiling_on_sc=False),
)
def _sc_pool(xt_hbm, emb_hbm, out_hbm, idx_v, acc_v, sem):
    wid = lax.axis_index("s") * 2 + lax.axis_index("c")
    col = wid * SPW
    pltpu.sync_copy(xt_hbm.at[:, pl.ds(col, SPW)], idx_v)

    pltpu.sync_copy(emb_hbm.at[idx_v.at[0]], acc_v)

    @pl.loop(0, L - 1)
    def _fire(i):
        pltpu.async_copy(emb_hbm.at[idx_v.at[i + 1]], acc_v, sem, add=True)

        @pl.when(i >= WINDOW - 1)
        def _():
            pltpu.make_async_copy(emb_hbm.at[idx_v.at[0]], acc_v, sem).wait()

    @pl.loop(0, WINDOW - 1)
    def _drain(_):
        pltpu.make_async_copy(emb_hbm.at[idx_v.at[0]], acc_v, sem).wait()

    pltpu.sync_copy(acc_v, out_hbm.at[pl.ds(wid * SPW, SPW)])


def _mlp_body(s_ref, w1_ref, b1_ref, w2_ref, b2_ref, o_ref):
    x = s_ref[...] * np.float32(1.0 / L)
    h = lax.dot_general(x, w1_ref[...], (((1,), (1,)), ((), ())),
                        preferred_element_type=jnp.float32)
    h = jnp.maximum(h + b1_ref[...], 0.0)
    o_ref[...] = lax.dot_general(h, w2_ref[...], (((1,), (1,)), ((), ())),
                                 preferred_element_type=jnp.float32) + b2_ref[...]


def _mlp(sums, W1, b1, W2, b2):
    return pl.pallas_call(
        _mlp_body,
        out_shape=jax.ShapeDtypeStruct((B, NCLS), jnp.float32),
    )(sums, W1, b1.reshape(1, HID), W2, b2.reshape(1, NCLS))


def kernel(x_in, emb, W1, b1, W2, b2):
    x_t = jnp.transpose(x_in)
    sums = _sc_pool(x_t, emb)
    return _mlp(sums, W1, b1, W2, b2)
